# baseline (speedup 1.0000x reference)
import math

import jax
import jax.numpy as jnp
from jax import lax
from jax.experimental import pallas as pl
from jax.experimental.pallas import tpu as pltpu

_N_CHUNKS = 8


def _fused_attn_kernel(z_hbm, ctx_ref, embd_ref, out_ref, z_vmem, sem):
    B, F = z_hbm.shape
    d = embd_ref.shape[1]
    S = B // _N_CHUNKS
    copies = []
    for c in range(_N_CHUNKS):
        cp = pltpu.make_async_copy(
            z_hbm.at[pl.ds(c * S, S)], z_vmem.at[pl.ds(c * S, S)], sem.at[c])
        cp.start()
        copies.append(cp)
    embd = embd_ref[...]
    k = math.log2(math.e) / math.sqrt(d)
    raws = []
    for c in range(_N_CHUNKS):
        raws.append(lax.dot_general(
            ctx_ref[pl.ds(c * S, S), :], embd, (((1,), (1,)), ((), ())),
            preferred_element_type=jnp.float32,
        ))
    for c in range(_N_CHUNKS):
        raw = raws[c]
        row_max = jnp.max(raw, axis=1, keepdims=True)
        copies[c].wait()
        mf = (z_vmem[pl.ds(c * S, S), :] > 0).astype(jnp.float32)
        ex = jnp.exp2((raw - row_max) * k) * mf
        denom = jnp.sum(ex, axis=1, keepdims=True)
        denom = jnp.where(denom == 0.0, 1.0, denom)
        counts = jnp.maximum(jnp.sum(mf, axis=1, keepdims=True), 1.0)
        acc = jnp.dot(ex, embd, preferred_element_type=jnp.float32)
        out_ref[pl.ds(c * S, S), :] = acc / (denom * counts)


def kernel(z_sparse, context_embedding, embd_weight):
    B, F = z_sparse.shape
    d = embd_weight.shape[1]
    return pl.pallas_call(
        _fused_attn_kernel,
        in_specs=[
            pl.BlockSpec(memory_space=pltpu.MemorySpace.HBM),
            pl.BlockSpec((B, d), lambda: (0, 0)),
            pl.BlockSpec((F, d), lambda: (0, 0)),
        ],
        out_specs=pl.BlockSpec((B, d), lambda: (0, 0)),
        out_shape=jax.ShapeDtypeStruct((B, d), jnp.float32),
        scratch_shapes=[
            pltpu.VMEM((B, F), jnp.float32),
            pltpu.SemaphoreType.DMA((_N_CHUNKS,)),
        ],
    )(z_sparse, context_embedding, embd_weight)


# speedup vs baseline: 1.0382x; 1.0382x over previous
import math

import jax
import jax.numpy as jnp
from jax import lax
from jax.experimental import pallas as pl
from jax.experimental.pallas import tpu as pltpu

_N_CHUNKS = 4


def _fused_attn_kernel(z_hbm, ctx_ref, embd_ref, out_ref, z_vmem, sem):
    B, F = z_hbm.shape
    d = embd_ref.shape[1]
    S = B // _N_CHUNKS
    copies = []
    for c in range(_N_CHUNKS):
        cp = pltpu.make_async_copy(
            z_hbm.at[pl.ds(c * S, S)], z_vmem.at[pl.ds(c * S, S)], sem.at[c])
        cp.start()
        copies.append(cp)
    embd = embd_ref[...]
    k = math.log2(math.e) / math.sqrt(d)
    raws = []
    for c in range(_N_CHUNKS):
        raws.append(lax.dot_general(
            ctx_ref[pl.ds(c * S, S), :], embd, (((1,), (1,)), ((), ())),
            preferred_element_type=jnp.float32,
        ))
    for c in range(_N_CHUNKS):
        raw = raws[c]
        row_max = jnp.max(raw, axis=1, keepdims=True)
        copies[c].wait()
        mf = (z_vmem[pl.ds(c * S, S), :] > 0).astype(jnp.float32)
        ex = jnp.exp2((raw - row_max) * k) * mf
        denom = jnp.sum(ex, axis=1, keepdims=True)
        denom = jnp.where(denom == 0.0, 1.0, denom)
        counts = jnp.maximum(jnp.sum(mf, axis=1, keepdims=True), 1.0)
        acc = jnp.dot(ex, embd, preferred_element_type=jnp.float32)
        out_ref[pl.ds(c * S, S), :] = acc / (denom * counts)


def kernel(z_sparse, context_embedding, embd_weight):
    B, F = z_sparse.shape
    d = embd_weight.shape[1]
    return pl.pallas_call(
        _fused_attn_kernel,
        in_specs=[
            pl.BlockSpec(memory_space=pltpu.MemorySpace.HBM),
            pl.BlockSpec((B, d), lambda: (0, 0)),
            pl.BlockSpec((F, d), lambda: (0, 0)),
        ],
        out_specs=pl.BlockSpec((B, d), lambda: (0, 0)),
        out_shape=jax.ShapeDtypeStruct((B, d), jnp.float32),
        scratch_shapes=[
            pltpu.VMEM((B, F), jnp.float32),
            pltpu.SemaphoreType.DMA((_N_CHUNKS,)),
        ],
    )(z_sparse, context_embedding, embd_weight)
